# baseline (speedup 1.0000x reference)
import functools

import jax
import jax.numpy as jnp
import numpy as np
from jax import lax
from jax.experimental import pallas as pl
from jax.experimental.pallas import tpu as pltpu
from jax.experimental.pallas import tpu_sc as plsc

_N_PART = 8388608
_N_BINS = 256
_LO, _HI = -6.0, 6.0
_BIN_W = (_HI - _LO) / _N_BINS
_INV_W = 1.0 / _BIN_W
_NOISE_SCALE = 0.05
_SEED = 0

_NC, _NS, _L = 2, 16, 16
_NW = _NC * _NS
_PER_W = _N_PART // _NW
_CHUNK = 16384
_NBUF = 4
_NCHUNK = _PER_W // _CHUNK
_HC = _N_BINS + 1
_HW = _L * _HC

_SCALE = np.asarray(
    jax.random.normal(jax.random.key(_SEED), (_N_BINS,), jnp.float32)
) * np.float32(_NOISE_SCALE)
_SCALE = (1.0 + _SCALE.astype(np.float32)) / np.float32(_N_PART * _BIN_W)
_SCALE = np.concatenate([_SCALE, np.zeros(_HC - _N_BINS, np.float32)])

_mesh = plsc.VectorSubcoreMesh(core_axis_name="c", subcore_axis_name="s")


@functools.partial(
    pl.kernel,
    out_type=jax.ShapeDtypeStruct((_NW, 2 * _HW), jnp.float32),
    mesh=_mesh,
    compiler_params=pltpu.CompilerParams(
        needs_layout_passes=False,
        use_tc_tiling_on_sc=True,
    ),
    scratch_types=[
        *([pltpu.VMEM((_CHUNK,), jnp.float32)] * _NBUF),
        pltpu.VMEM((2 * _HW,), jnp.float32),
        *([pltpu.SemaphoreType.DMA] * _NBUF),
    ],
)
def _hist_sc(xt, out, *refs):
    bufs = refs[:_NBUF]
    hist = refs[_NBUF]
    sems = refs[_NBUF + 1 : 2 * _NBUF + 1]
    cid = lax.axis_index("c")
    sid = lax.axis_index("s")
    wid = sid * _NC + cid
    base = wid * _PER_W

    lane = lax.iota(jnp.int32, _L)
    laneoff = (lane * _HC).astype(jnp.float32)
    cvec = laneoff + (-_LO * _INV_W)
    lovec = laneoff
    hivec = laneoff + float(_N_BINS - 1)
    zf = laneoff * 0.0

    @plsc.parallel_loop(0, 2 * _HW // _L, unroll=2)
    def zero(j):
        hist[pl.ds(j * _L, _L)] = zf

    def start(ci, b):
        pltpu.async_copy(
            xt.at[0, pl.ds(base + ci * _CHUNK, _CHUNK)], bufs[b], sems[b]
        )

    def wait(b):
        pltpu.make_async_copy(
            xt.at[0, pl.ds(base, _CHUNK)], bufs[b], sems[b]
        ).wait()

    def process(b):
        buf = bufs[b]

        @plsc.parallel_loop(0, _CHUNK // _L, unroll=8)
        def body(it):
            v = buf[pl.ds(it * _L, _L)]
            u = jnp.minimum(jnp.maximum(v * _INV_W + cvec, lovec), hivec)
            reg = (it & 1) * _HW
            i0 = u.astype(jnp.int32) + reg
            f = u - i0.astype(jnp.float32) + reg.astype(jnp.float32)
            plsc.addupdate_scatter(hist, [i0], 1.0 - f)
            plsc.addupdate_scatter(hist, [i0 + 1], f)

    for b in range(_NBUF):
        start(b, b)

    def outer(g, carry):
        for b in range(_NBUF):
            wait(b)
            process(b)

            @pl.when(g * _NBUF + b + _NBUF < _NCHUNK)
            def _():
                start(g * _NBUF + b + _NBUF, b)

        return carry

    lax.fori_loop(0, _NCHUNK // _NBUF, outer, 0)

    pltpu.sync_copy(hist, out.at[wid])


def _finish_body(parts_ref, scale_ref, o_ref):
    s = jnp.sum(parts_ref[...], axis=0)
    o_ref[...] = jnp.maximum(s * scale_ref[...], 0.0)


_finish = pl.pallas_call(
    _finish_body,
    out_shape=jax.ShapeDtypeStruct((_HC,), jnp.float32),
)


def kernel(x):
    xt = x.T
    parts = _hist_sc(xt)
    out = _finish(parts.reshape(_NW * 2 * _L, _HC), jnp.asarray(_SCALE))
    return out[:_N_BINS]
